# baseline (speedup 1.0000x reference)
import math

import jax
import jax.numpy as jnp
from jax import lax
from jax.experimental import pallas as pl
from jax.experimental.pallas import tpu as pltpu
from jax.experimental.pallas import tpu_sc as plsc

_NUM_LEVELS = 16
_F = 2
_T = 2 ** 19
_MASK = _T - 1
_BASE_RES = 16
_FINEST_RES = 2048
_GROWTH = math.exp(
    (math.log(_FINEST_RES) - math.log(_BASE_RES)) / (_NUM_LEVELS - 1))
_RES = [max(1, int(round(_BASE_RES * _GROWTH ** l)))
        for l in range(_NUM_LEVELS)]
_N = 262144
_NC, _NS = 2, 16
_NW = _NC * _NS
_PW = _N // _NW
_C = 128
_NCHUNK = _PW // _C
_G = _C // 16
_PA = 73856093
_PB = 19349663
_PC = 83492791


def _body(x_hbm, table_hbm, out_hbm, x_v, idx_v, off_v, rows_v, out_v, sem):
    wid = lax.axis_index("s") * _NC + lax.axis_index("c")
    wid32 = lax.convert_element_type(wid, jnp.int32)
    lane = lax.iota(jnp.int32, 16)
    z16 = jnp.zeros((16,), jnp.int32)
    o16 = jnp.full((16,), 1, jnp.int32)
    t16 = jnp.full((16,), 2, jnp.int32)

    def chunk_body(ci, carry):
        pbase = wid32 * jnp.int32(_PW) + ci * jnp.int32(_C)
        pltpu.sync_copy(x_hbm.at[pl.ds(pbase, _C)], x_v)

        for l in range(_NUM_LEVELS):
            res = float(_RES[l])
            lvl_off = l * _T

            def grp_idx(g, c2, l=l, res=res, lvl_off=lvl_off):
                pidx = g * jnp.int32(16) + lane
                xs0 = plsc.load_gather(x_v, [pidx, z16]) * res
                xs1 = plsc.load_gather(x_v, [pidx, o16]) * res
                xs2 = plsc.load_gather(x_v, [pidx, t16]) * res
                i0 = xs0.astype(jnp.int32)
                i1 = xs1.astype(jnp.int32)
                i2 = xs2.astype(jnp.int32)
                a0 = i0 * jnp.int32(_PA)
                a1 = a0 + jnp.int32(_PA)
                b0 = i1 * jnp.int32(_PB)
                b1 = b0 + jnp.int32(_PB)
                c0 = i2 * jnp.int32(_PC)
                c1 = c0 + jnp.int32(_PC)
                corner = 0
                for aa in (a0, a1):
                    for bb in (b0, b1):
                        for cc in (c0, c1):
                            h = ((aa ^ bb ^ cc) & jnp.int32(_MASK)) \
                                + jnp.int32(lvl_off)
                            idx_v[jnp.int32(corner),
                                  pl.ds(g * jnp.int32(16), 16)] = \
                                lax.shift_right_logical(h, jnp.int32(3))
                            off_v[jnp.int32(corner),
                                  pl.ds(g * jnp.int32(16), 16)] = \
                                lax.shift_left(h & jnp.int32(7), jnp.int32(1))
                            corner += 1
                return c2

            lax.fori_loop(jnp.int32(0), jnp.int32(_G), grp_idx,
                          jnp.int32(0))

            handles = [
                pltpu.async_copy(table_hbm.at[idx_v.at[jnp.int32(c)]],
                                 rows_v.at[jnp.int32(c)], sem)
                for c in range(8)
            ]
            for h in handles:
                h.wait()

            def grp_comb(g, c2, l=l, res=res):
                pidx = g * jnp.int32(16) + lane
                xs0 = plsc.load_gather(x_v, [pidx, z16]) * res
                xs1 = plsc.load_gather(x_v, [pidx, o16]) * res
                xs2 = plsc.load_gather(x_v, [pidx, t16]) * res
                w0 = xs0 - xs0.astype(jnp.int32).astype(jnp.float32)
                w1 = xs1 - xs1.astype(jnp.int32).astype(jnp.float32)
                w2 = xs2 - xs2.astype(jnp.int32).astype(jnp.float32)
                u0 = 1.0 - w0
                u1 = 1.0 - w1
                u2 = 1.0 - w2
                p00 = u0 * u1
                p01 = u0 * w1
                p10 = w0 * u1
                p11 = w0 * w1
                wts = (p00 * u2, p00 * w2, p01 * u2, p01 * w2,
                       p10 * u2, p10 * w2, p11 * u2, p11 * w2)
                acc0 = None
                acc1 = None
                for c in range(8):
                    csp = jnp.full((16,), c, jnp.int32)
                    off = off_v[jnp.int32(c), pl.ds(g * jnp.int32(16), 16)]
                    f0 = plsc.load_gather(rows_v, [csp, pidx, off])
                    f1 = plsc.load_gather(rows_v, [csp, pidx, off + o16])
                    t0 = wts[c] * f0
                    t1 = wts[c] * f1
                    acc0 = t0 if acc0 is None else acc0 + t0
                    acc1 = t1 if acc1 is None else acc1 + t1
                pat = lane * jnp.int32(32) + (
                    g * jnp.int32(512) + jnp.int32(2 * l))
                plsc.store_scatter(out_v, [pat], acc0)
                plsc.store_scatter(out_v, [pat + jnp.int32(1)], acc1)
                return c2

            lax.fori_loop(jnp.int32(0), jnp.int32(_G), grp_comb,
                          jnp.int32(0))

        pltpu.sync_copy(out_v, out_hbm.at[pl.ds(pbase * jnp.int32(32),
                                                _C * 32)])
        return carry

    lax.fori_loop(jnp.int32(0), jnp.int32(_NCHUNK), chunk_body,
                  jnp.int32(0))


def kernel(x, tables):
    mesh = plsc.VectorSubcoreMesh(
        core_axis_name="c", subcore_axis_name="s",
        num_cores=_NC, num_subcores=_NS)
    k = pl.kernel(
        _body,
        out_type=jax.ShapeDtypeStruct((_N * _NUM_LEVELS * _F,), jnp.float32),
        mesh=mesh,
        scratch_types=[
            pltpu.VMEM((_C, 3), jnp.float32),
            pltpu.VMEM((8, _C), jnp.int32),
            pltpu.VMEM((8, _C), jnp.int32),
            pltpu.VMEM((8, _C, 16), jnp.float32),
            pltpu.VMEM((_C * _NUM_LEVELS * _F,), jnp.float32),
            pltpu.SemaphoreType.DMA,
        ],
        compiler_params=pltpu.CompilerParams(
            needs_layout_passes=False, use_tc_tiling_on_sc=False),
    )
    table_flat = tables.reshape(_NUM_LEVELS * _T * _F // 16, 16) \
        .astype(jnp.float32)
    out = k(x.astype(jnp.float32), table_flat)
    return out.reshape(_N, _NUM_LEVELS, _F)


# speedup vs baseline: 1.0752x; 1.0752x over previous
import math

import jax
import jax.numpy as jnp
from jax import lax
from jax.experimental import pallas as pl
from jax.experimental.pallas import tpu as pltpu
from jax.experimental.pallas import tpu_sc as plsc

_NUM_LEVELS = 16
_F = 2
_T = 2 ** 19
_MASK = _T - 1
_BASE_RES = 16
_FINEST_RES = 2048
_GROWTH = math.exp(
    (math.log(_FINEST_RES) - math.log(_BASE_RES)) / (_NUM_LEVELS - 1))
_RES = [max(1, int(round(_BASE_RES * _GROWTH ** l)))
        for l in range(_NUM_LEVELS)]
_N = 262144
_NC, _NS = 2, 16
_NW = _NC * _NS
_PW = _N // _NW
_C = 128
_NCHUNK = _PW // _C
_G = _C // 16
_PA = 73856093
_PB = 19349663
_PC = 83492791


def _body(x_hbm, table_hbm, out_hbm, x_v,
          idx0, idx1, off0, off1, rows0, rows1, out_v, sem0, sem1):
    wid = lax.axis_index("s") * _NC + lax.axis_index("c")
    wid32 = lax.convert_element_type(wid, jnp.int32)
    lane = lax.iota(jnp.int32, 16)
    z16 = jnp.zeros((16,), jnp.int32)
    o16 = jnp.full((16,), 1, jnp.int32)
    t16 = jnp.full((16,), 2, jnp.int32)
    idx_b = (idx0, idx1)
    off_b = (off0, off1)
    rows_b = (rows0, rows1)
    sem_b = (sem0, sem1)

    def chunk_body(ci, carry):
        pbase = wid32 * jnp.int32(_PW) + ci * jnp.int32(_C)
        pltpu.sync_copy(x_hbm.at[pl.ds(pbase, _C)], x_v)

        def fire(l):
            b = l & 1
            res = float(_RES[l])
            lvl_off = l * _T
            idx_v, off_v = idx_b[b], off_b[b]

            def grp_idx(g, c2):
                pidx = g * jnp.int32(16) + lane
                xs0 = plsc.load_gather(x_v, [pidx, z16]) * res
                xs1 = plsc.load_gather(x_v, [pidx, o16]) * res
                xs2 = plsc.load_gather(x_v, [pidx, t16]) * res
                i0 = xs0.astype(jnp.int32)
                i1 = xs1.astype(jnp.int32)
                i2 = xs2.astype(jnp.int32)
                a0 = i0 * jnp.int32(_PA)
                a1 = a0 + jnp.int32(_PA)
                b0 = i1 * jnp.int32(_PB)
                b1 = b0 + jnp.int32(_PB)
                c0 = i2 * jnp.int32(_PC)
                c1 = c0 + jnp.int32(_PC)
                corner = 0
                for aa in (a0, a1):
                    for bb in (b0, b1):
                        for cc in (c0, c1):
                            h = ((aa ^ bb ^ cc) & jnp.int32(_MASK)) \
                                + jnp.int32(lvl_off)
                            idx_v[jnp.int32(corner),
                                  pl.ds(g * jnp.int32(16), 16)] = \
                                lax.shift_right_logical(h, jnp.int32(3))
                            off_v[jnp.int32(corner),
                                  pl.ds(g * jnp.int32(16), 16)] = \
                                lax.shift_left(h & jnp.int32(7), jnp.int32(1))
                            corner += 1
                return c2

            lax.fori_loop(jnp.int32(0), jnp.int32(_G), grp_idx, jnp.int32(0))
            return [
                pltpu.async_copy(table_hbm.at[idx_v.at[jnp.int32(c)]],
                                 rows_b[b].at[jnp.int32(c)], sem_b[b])
                for c in range(8)
            ]

        def combine(l):
            b = l & 1
            res = float(_RES[l])
            off_v, rows_v = off_b[b], rows_b[b]

            def grp_comb(g, c2):
                pidx = g * jnp.int32(16) + lane
                xs0 = plsc.load_gather(x_v, [pidx, z16]) * res
                xs1 = plsc.load_gather(x_v, [pidx, o16]) * res
                xs2 = plsc.load_gather(x_v, [pidx, t16]) * res
                w0 = xs0 - xs0.astype(jnp.int32).astype(jnp.float32)
                w1 = xs1 - xs1.astype(jnp.int32).astype(jnp.float32)
                w2 = xs2 - xs2.astype(jnp.int32).astype(jnp.float32)
                u0 = 1.0 - w0
                u1 = 1.0 - w1
                u2 = 1.0 - w2
                p00 = u0 * u1
                p01 = u0 * w1
                p10 = w0 * u1
                p11 = w0 * w1
                wts = (p00 * u2, p00 * w2, p01 * u2, p01 * w2,
                       p10 * u2, p10 * w2, p11 * u2, p11 * w2)
                acc0 = None
                acc1 = None
                for c in range(8):
                    csp = jnp.full((16,), c, jnp.int32)
                    off = off_v[jnp.int32(c), pl.ds(g * jnp.int32(16), 16)]
                    f0 = plsc.load_gather(rows_v, [csp, pidx, off])
                    f1 = plsc.load_gather(rows_v, [csp, pidx, off + o16])
                    t0 = wts[c] * f0
                    t1 = wts[c] * f1
                    acc0 = t0 if acc0 is None else acc0 + t0
                    acc1 = t1 if acc1 is None else acc1 + t1
                pat = lane * jnp.int32(32) + (
                    g * jnp.int32(512) + jnp.int32(2 * l))
                plsc.store_scatter(out_v, [pat], acc0)
                plsc.store_scatter(out_v, [pat + jnp.int32(1)], acc1)
                return c2

            lax.fori_loop(jnp.int32(0), jnp.int32(_G), grp_comb, jnp.int32(0))

        handles = fire(0)
        for l in range(_NUM_LEVELS):
            nxt = fire(l + 1) if l + 1 < _NUM_LEVELS else None
            for h in handles:
                h.wait()
            combine(l)
            handles = nxt

        pltpu.sync_copy(out_v, out_hbm.at[pl.ds(pbase * jnp.int32(32),
                                                _C * 32)])
        return carry

    lax.fori_loop(jnp.int32(0), jnp.int32(_NCHUNK), chunk_body,
                  jnp.int32(0))


def kernel(x, tables):
    mesh = plsc.VectorSubcoreMesh(
        core_axis_name="c", subcore_axis_name="s",
        num_cores=_NC, num_subcores=_NS)
    k = pl.kernel(
        _body,
        out_type=jax.ShapeDtypeStruct((_N * _NUM_LEVELS * _F,), jnp.float32),
        mesh=mesh,
        scratch_types=[
            pltpu.VMEM((_C, 3), jnp.float32),
            pltpu.VMEM((8, _C), jnp.int32),
            pltpu.VMEM((8, _C), jnp.int32),
            pltpu.VMEM((8, _C), jnp.int32),
            pltpu.VMEM((8, _C), jnp.int32),
            pltpu.VMEM((8, _C, 16), jnp.float32),
            pltpu.VMEM((8, _C, 16), jnp.float32),
            pltpu.VMEM((_C * _NUM_LEVELS * _F,), jnp.float32),
            pltpu.SemaphoreType.DMA,
            pltpu.SemaphoreType.DMA,
        ],
        compiler_params=pltpu.CompilerParams(
            needs_layout_passes=False, use_tc_tiling_on_sc=False),
    )
    table_flat = tables.reshape(_NUM_LEVELS * _T * _F // 16, 16) \
        .astype(jnp.float32)
    out = k(x.astype(jnp.float32), table_flat)
    return out.reshape(_N, _NUM_LEVELS, _F)
